# baseline (speedup 1.0000x reference)
import functools

import jax
import jax.numpy as jnp
from jax import lax
from jax.experimental import pallas as pl
from jax.experimental.pallas import tpu as pltpu
from jax.experimental.pallas import tpu_sc as plsc

_LAMBDA = 0.5
_K = 32
_TAU = 0.8
_B = 128
_N = 8192
_BLK = 8
_GRID = _B // _BLK
_IMIN = -2147483648
_NC = 2
_NS = 16
_NW = _NC * _NS
_RPW = _B // _NW
_NBINS = 2048
_CHUNKS = _N // 16


def _sortable_key(v_f32):
    b = plsc.bitcast(v_f32, jnp.int32)
    return b ^ lax.shift_right_logical(jnp.right_shift(b, 31), 1)


def _scalar(x):
    if x.ndim == 0:
        return x
    return lax.squeeze(lax.slice(x, (0,), (1,)), (0,))


def _sc_thresh_body(g_hbm, out_hbm, g_v, hist, coll, outv):
    wid = lax.axis_index("s") * _NC + lax.axis_index("c")
    base_row = wid * _RPW
    pltpu.sync_copy(g_hbm.at[pl.ds(base_row, _RPW)], g_v)

    lanes = lax.iota(jnp.int32, 16)
    big = jnp.int32(2147483647)

    for r in range(_RPW):
        def _zero(i, carry):
            hist[pl.ds(i * 16, 16)] = jnp.zeros((16,), jnp.int32)
            return carry

        lax.fori_loop(0, _NBINS // 16, _zero, 0)

        def _hist(i, carry):
            gv = g_v[r, pl.ds(i * 16, 16)]
            key = _sortable_key(gv)
            binidx = lax.shift_right_logical(key ^ jnp.int32(_IMIN), 21)
            plsc.addupdate_scatter(hist, [binidx], jnp.ones((16,), jnp.int32))
            return carry

        lax.fori_loop(0, _CHUNKS, _hist, 0)

        def _scan(i, carry):
            run_total, nle, minv = carry
            h = hist[pl.ds(i * 16, 16)]
            c = plsc.cumsum(h) + run_total
            nle = nle + _scalar(
                plsc.all_reduce_population_count(c <= jnp.int32(_N - _K)))
            cand = jnp.where(c > jnp.int32(_N - _K), c, big)
            minv = jnp.minimum(minv, _scalar(lax.reduce_min(cand, (0,))))
            run_total = run_total + _scalar(lax.reduce_sum(h, (0,)))
            return run_total, nle, minv

        _, cutbin, min_cum = lax.fori_loop(
            0, _NBINS // 16, _scan, (jnp.int32(0), jnp.int32(0), big))
        cnt_above = jnp.int32(_N) - min_cum

        def _collect(i, off):
            gv = g_v[r, pl.ds(i * 16, 16)]
            key = _sortable_key(gv)
            binidx = lax.shift_right_logical(key ^ jnp.int32(_IMIN), 21)
            eq = binidx == cutbin
            plsc.store_compressed(coll.at[pl.ds(off, 16)], key, mask=eq)
            return off + _scalar(plsc.all_reduce_population_count(eq))

        off = lax.fori_loop(0, _CHUNKS, _collect, jnp.int32(0))
        coll[pl.ds(off, 16)] = jnp.full((16,), jnp.int32(_IMIN))

        a0 = plsc.sort_key_val(coll[pl.ds(0, 16)], coll[pl.ds(0, 16)],
                               descending=True)[0]
        b0 = jnp.full((16,), jnp.int32(_IMIN))

        def _mcond(carry):
            j, _, _ = carry
            return j * 16 < off

        def _mbody(carry):
            j, a, b = carry
            cs = plsc.sort_key_val(coll[pl.ds(j * 16, 16)],
                                   coll[pl.ds(j * 16, 16)],
                                   descending=False)[0]
            hi = jnp.maximum(a, cs)
            lo = jnp.minimum(a, cs)
            a2 = plsc.sort_key_val(hi, hi, descending=True)[0]
            lo_s = plsc.sort_key_val(lo, lo, descending=False)[0]
            hi2 = jnp.maximum(b, lo_s)
            b2 = plsc.sort_key_val(hi2, hi2, descending=True)[0]
            return j + 1, a2, b2

        _, a_top, b_top = lax.while_loop(_mcond, _mbody,
                                         (jnp.int32(1), a0, b0))

        ri = (jnp.int32(_K) - cnt_above) - 1
        la = _scalar(lax.reduce_max(
            jnp.where(lanes == ri, a_top, jnp.int32(_IMIN)), (0,)))
        lb = _scalar(lax.reduce_max(
            jnp.where(lanes == ri - 16, b_top, jnp.int32(_IMIN)), (0,)))
        thr_key = jnp.where(ri < 16, la, lb)
        thr_bits = (thr_key
                    ^ lax.shift_right_logical(jnp.right_shift(thr_key, 31), 1))
        thr_f = plsc.bitcast(jnp.broadcast_to(thr_bits, (16,)), jnp.float32)

        ov = outv[...]
        outv[...] = jnp.where(lanes == r, thr_f, ov)

    pltpu.sync_copy(outv, out_hbm.at[wid])


def _sc_thresholds(gamma):
    mesh = plsc.VectorSubcoreMesh(core_axis_name="c", subcore_axis_name="s")
    run = pl.kernel(
        _sc_thresh_body,
        out_type=jax.ShapeDtypeStruct((_NW, 16), jnp.float32),
        mesh=mesh,
        scratch_types=[
            pltpu.VMEM((_RPW, _N), jnp.float32),
            pltpu.VMEM((_NBINS,), jnp.int32),
            pltpu.VMEM((_N + 32,), jnp.int32),
            pltpu.VMEM((16,), jnp.float32),
        ],
        compiler_params=pltpu.CompilerParams(needs_layout_passes=False),
    )
    out = run(gamma)
    return out[:, :_RPW].reshape(_B, 1)


def _loss_body(p_ref, r_ref, g_ref, t_ref, out_ref, acc_ref):
    i = pl.program_id(0)

    @pl.when(i == 0)
    def _init():
        acc_ref[0] = 0.0
        acc_ref[1] = 0.0
        acc_ref[2] = 0.0
        acc_ref[3] = 0.0

    g = g_ref[...]
    p = p_ref[...]
    inv_tau = jnp.float32(1.0 / _TAU)

    gmax = jnp.max(g, axis=1, keepdims=True)
    pmax = jnp.max(p, axis=1, keepdims=True)
    eg = jnp.exp((g - gmax) * inv_tau)
    ep = jnp.exp((p - pmax) * inv_tau)
    zg = jnp.sum(eg, axis=1, keepdims=True)
    zp = jnp.sum(ep, axis=1, keepdims=True)
    s_raw = jnp.sum(eg * (g - p), axis=1, keepdims=True)

    mask = g >= t_ref[...]
    cnt_row = jnp.sum(mask.astype(jnp.float32), axis=1, keepdims=True)
    d = p - g
    mse = jnp.sum(jnp.where(mask, d * d, jnp.float32(0.0)))

    link = jnp.sum((r_ref[...] - gmax) ** 2)
    kl = jnp.sum(s_raw / (zg * _TAU) + (pmax - gmax) * inv_tau
                 + jnp.log(zp / zg))

    acc_ref[0] += mse
    acc_ref[1] += jnp.sum(cnt_row)
    acc_ref[2] += link
    acc_ref[3] += kl

    @pl.when(i == _GRID - 1)
    def _fin():
        total = (acc_ref[0] / jnp.maximum(acc_ref[1], 1.0)
                 + _LAMBDA * acc_ref[2] / _B
                 + (_TAU * _TAU / _B) * acc_ref[3])
        out_ref[...] = total.reshape((1, 1))


@jax.jit
def kernel(pred_logits, rsrp_pred, gamma_true):
    thr = _sc_thresholds(gamma_true)
    out = pl.pallas_call(
        _loss_body,
        grid=(_GRID,),
        in_specs=[
            pl.BlockSpec((_BLK, _N), lambda i: (i, 0)),
            pl.BlockSpec((_BLK, 1), lambda i: (i, 0)),
            pl.BlockSpec((_BLK, _N), lambda i: (i, 0)),
            pl.BlockSpec((_BLK, 1), lambda i: (i, 0)),
        ],
        out_specs=pl.BlockSpec((1, 1), lambda i: (0, 0)),
        out_shape=jax.ShapeDtypeStruct((1, 1), jnp.float32),
        scratch_shapes=[pltpu.SMEM((4,), jnp.float32)],
    )(pred_logits, rsrp_pred, gamma_true, thr)
    return out[0, 0]


# speedup vs baseline: 2.0233x; 2.0233x over previous
import functools

import jax
import jax.numpy as jnp
from jax import lax
from jax.experimental import pallas as pl
from jax.experimental.pallas import tpu as pltpu
from jax.experimental.pallas import tpu_sc as plsc

_LAMBDA = 0.5
_K = 32
_TAU = 0.8
_B = 128
_N = 8192
_BLK = 32
_GRID = _B // _BLK
_IMIN = -2147483648
_NC = 2
_NS = 16
_NW = _NC * _NS
_RPW = _B // _NW
_NBINS = 2048
_CHUNKS = _N // 16


def _sortable_key(v_f32):
    b = plsc.bitcast(v_f32, jnp.int32)
    return b ^ lax.shift_right_logical(jnp.right_shift(b, 31), 1)


def _scalar(x):
    if x.ndim == 0:
        return x
    return lax.squeeze(lax.slice(x, (0,), (1,)), (0,))


def _sc_mse_body(g_hbm, p_hbm, out_hbm, g_v, p_v, hist, collk, collv, outv):
    wid = lax.axis_index("s") * _NC + lax.axis_index("c")
    base_row = wid * _RPW
    pltpu.sync_copy(g_hbm.at[pl.ds(base_row, _RPW)], g_v)
    pltpu.sync_copy(p_hbm.at[pl.ds(base_row, _RPW)], p_v)

    lanes = lax.iota(jnp.int32, 16)
    ones = jnp.ones((16,), jnp.int32)
    msew = jnp.float32(0.0)

    for r in range(_RPW):
        @plsc.parallel_loop(0, _NBINS // 16, unroll=8)
        def _zero(i):
            hist[pl.ds(i * 16, 16)] = jnp.zeros((16,), jnp.int32)

        @plsc.parallel_loop(0, _CHUNKS, unroll=8)
        def _hist(i):
            gv = g_v[r, pl.ds(i * 16, 16)]
            key = _sortable_key(gv)
            binidx = lax.shift_right_logical(key ^ jnp.int32(_IMIN), 21)
            plsc.addupdate_scatter(hist, [binidx], ones)

        def _scond(c):
            return jnp.logical_not(c[4])

        def _sbody(c):
            i, tot, cutbin, cntab, _ = c
            h = hist[pl.ds(i * 16, 16)]
            cs = plsc.cumsum(h)
            ctot = _scalar(lax.reduce_max(cs, (0,)))
            a = (tot + ctot - cs) + h
            ge = a >= jnp.int32(_K)
            npos = _scalar(plsc.all_reduce_population_count(ge))
            crossed = npos > 0
            lstar = npos - 1
            al = _scalar(lax.reduce_max(
                jnp.where(lanes == lstar, a, 0), (0,)))
            hl = _scalar(lax.reduce_max(
                jnp.where(lanes == lstar, h, 0), (0,)))
            return (i - 1, tot + ctot,
                    jnp.where(crossed, i * 16 + lstar, cutbin),
                    jnp.where(crossed, al - hl, cntab),
                    crossed)

        _, _, cutbin, cntab, _ = lax.while_loop(
            _scond, _sbody,
            (jnp.int32(_NBINS // 16 - 1), jnp.int32(0), jnp.int32(0),
             jnp.int32(0), jnp.bool_(False)))

        @plsc.parallel_loop(0, _CHUNKS, unroll=4,
                            carry=(jnp.int32(0),
                                   jnp.zeros((16,), jnp.float32)))
        def _collect(i, c):
            off, acc = c
            gv = g_v[r, pl.ds(i * 16, 16)]
            pv = p_v[r, pl.ds(i * 16, 16)]
            key = _sortable_key(gv)
            binidx = lax.shift_right_logical(key ^ jnp.int32(_IMIN), 21)
            d = pv - gv
            d2 = d * d
            acc = acc + jnp.where(binidx > cutbin, d2, jnp.float32(0.0))
            eq = binidx == cutbin
            plsc.store_compressed(collk.at[pl.ds(off, 16)], key, mask=eq)
            plsc.store_compressed(collv.at[pl.ds(off, 16)], d2, mask=eq)
            return off + _scalar(plsc.all_reduce_population_count(eq)), acc

        off, acc = _collect
        collk[pl.ds(off, 16)] = jnp.full((16,), jnp.int32(_IMIN))
        collv[pl.ds(off, 16)] = jnp.zeros((16,), jnp.float32)

        s0 = plsc.sort_key_val(collk[pl.ds(0, 16)], collv[pl.ds(0, 16)],
                               descending=True)
        ak, av = s0[0], s0[1]
        bk = jnp.full((16,), jnp.int32(_IMIN))
        bv = jnp.zeros((16,), jnp.float32)

        def _mcond(c):
            return c[0] * 16 < off

        def _mbody(c):
            j, ak, av, bk, bv = c
            s = plsc.sort_key_val(collk[pl.ds(j * 16, 16)],
                                  collv[pl.ds(j * 16, 16)],
                                  descending=False)
            ck, cv = s[0], s[1]
            wa = ak >= ck
            hk = jnp.where(wa, ak, ck)
            hv = jnp.where(wa, av, cv)
            lk = jnp.where(wa, ck, ak)
            lv = jnp.where(wa, cv, av)
            s1 = plsc.sort_key_val(hk, hv, descending=True)
            s2 = plsc.sort_key_val(lk, lv, descending=False)
            wb = bk >= s2[0]
            h2k = jnp.where(wb, bk, s2[0])
            h2v = jnp.where(wb, bv, s2[1])
            s3 = plsc.sort_key_val(h2k, h2v, descending=True)
            return j + 1, s1[0], s1[1], s3[0], s3[1]

        _, ak, av, bk, bv = lax.while_loop(
            _mcond, _mbody, (jnp.int32(1), ak, av, bk, bv))

        r_need = jnp.int32(_K) - cntab
        msecut = (_scalar(lax.reduce_sum(
                      jnp.where(lanes < r_need, av, 0.0), (0,)))
                  + _scalar(lax.reduce_sum(
                      jnp.where(lanes + 16 < r_need, bv, 0.0), (0,))))
        msew = msew + _scalar(lax.reduce_sum(acc, (0,))) + msecut

    outv[...] = jnp.where(lanes == 0, msew, jnp.float32(0.0))
    pltpu.sync_copy(outv, out_hbm.at[wid])


def _sc_mse_partials(gamma, pred):
    mesh = plsc.VectorSubcoreMesh(core_axis_name="c", subcore_axis_name="s")
    run = pl.kernel(
        _sc_mse_body,
        out_type=jax.ShapeDtypeStruct((_NW, 16), jnp.float32),
        mesh=mesh,
        scratch_types=[
            pltpu.VMEM((_RPW, _N), jnp.float32),
            pltpu.VMEM((_RPW, _N), jnp.float32),
            pltpu.VMEM((_NBINS,), jnp.int32),
            pltpu.VMEM((_N + 32,), jnp.int32),
            pltpu.VMEM((_N + 32,), jnp.float32),
            pltpu.VMEM((16,), jnp.float32),
        ],
        compiler_params=pltpu.CompilerParams(needs_layout_passes=False),
    )
    return run(gamma, pred)


def _stats_body(p_ref, r_ref, g_ref, out_ref, acc_ref):
    i = pl.program_id(0)

    @pl.when(i == 0)
    def _init():
        acc_ref[0] = 0.0
        acc_ref[1] = 0.0

    g = g_ref[...]
    p = p_ref[...]
    inv_tau = jnp.float32(1.0 / _TAU)

    gmax = jnp.max(g, axis=1, keepdims=True)
    pmax = jnp.max(p, axis=1, keepdims=True)
    eg = jnp.exp((g - gmax) * inv_tau)
    ep = jnp.exp((p - pmax) * inv_tau)
    zg = jnp.sum(eg, axis=1, keepdims=True)
    zp = jnp.sum(ep, axis=1, keepdims=True)
    s_raw = jnp.sum(eg * (g - p), axis=1, keepdims=True)

    link = jnp.sum((r_ref[...] - gmax) ** 2)
    kl = jnp.sum(s_raw / (zg * _TAU) + (pmax - gmax) * inv_tau
                 + jnp.log(zp / zg))

    acc_ref[0] += link
    acc_ref[1] += kl

    @pl.when(i == _GRID - 1)
    def _fin():
        lane4 = lax.broadcasted_iota(jnp.int32, (1, 4), 1)
        out_ref[...] = jnp.where(
            lane4 == 0, acc_ref[0],
            jnp.where(lane4 == 1, acc_ref[1], 0.0))


def _combine_body(sc_ref, st_ref, out_ref):
    mse = jnp.sum(sc_ref[...][:, 0:1])
    st = st_ref[...]
    link = st[0, 0]
    kl = st[0, 1]
    total = (mse / jnp.float32(_B * _K)
             + _LAMBDA * link / _B
             + (_TAU * _TAU / _B) * kl)
    out_ref[...] = total.reshape((1, 1))


@jax.jit
def kernel(pred_logits, rsrp_pred, gamma_true):
    sc_out = _sc_mse_partials(gamma_true, pred_logits)
    stats = pl.pallas_call(
        _stats_body,
        grid=(_GRID,),
        in_specs=[
            pl.BlockSpec((_BLK, _N), lambda i: (i, 0)),
            pl.BlockSpec((_BLK, 1), lambda i: (i, 0)),
            pl.BlockSpec((_BLK, _N), lambda i: (i, 0)),
        ],
        out_specs=pl.BlockSpec((1, 4), lambda i: (0, 0)),
        out_shape=jax.ShapeDtypeStruct((1, 4), jnp.float32),
        scratch_shapes=[pltpu.SMEM((2,), jnp.float32)],
    )(pred_logits, rsrp_pred, gamma_true)
    out = pl.pallas_call(
        _combine_body,
        out_shape=jax.ShapeDtypeStruct((1, 1), jnp.float32),
    )(sc_out, stats)
    return out[0, 0]


# speedup vs baseline: 2.0432x; 1.0098x over previous
import functools

import jax
import jax.numpy as jnp
from jax import lax
from jax.experimental import pallas as pl
from jax.experimental.pallas import tpu as pltpu
from jax.experimental.pallas import tpu_sc as plsc

_LAMBDA = 0.5
_K = 32
_TAU = 0.8
_B = 128
_N = 8192
_BLK = 32
_GRID = _B // _BLK
_IMIN = -2147483648
_NC = 2
_NS = 16
_NW = _NC * _NS
_RPW = _B // _NW
_NBINS = 2048
_CHUNKS = _N // 16


def _scalar(x):
    if x.ndim == 0:
        return x
    return lax.squeeze(lax.slice(x, (0,), (1,)), (0,))


def _sc_mse_body(g_hbm, p_hbm, out_hbm, g_v, p_v, hist, collk, collv, outv):
    wid = lax.axis_index("s") * _NC + lax.axis_index("c")
    base_row = wid * _RPW
    pltpu.sync_copy(g_hbm.at[pl.ds(base_row, _RPW)], g_v)
    pltpu.sync_copy(p_hbm.at[pl.ds(base_row, _RPW)], p_v)

    lanes = lax.iota(jnp.int32, 16)
    ones = jnp.ones((16,), jnp.int32)
    msew = jnp.float32(0.0)

    for r in range(_RPW):
        @plsc.parallel_loop(0, _NBINS // 16, unroll=8)
        def _zero(i):
            hist[pl.ds(i * 16, 16)] = jnp.zeros((16,), jnp.int32)

        @plsc.parallel_loop(0, _CHUNKS, unroll=8)
        def _hist(i):
            gv = g_v[r, pl.ds(i * 16, 16)]
            b = plsc.bitcast(gv, jnp.int32)
            ukey = b ^ (jnp.right_shift(b, 31) | jnp.int32(_IMIN))
            binidx = lax.shift_right_logical(ukey, 21)
            plsc.addupdate_scatter(hist, [binidx], ones)

        def _scond(c):
            return jnp.logical_not(c[4])

        def _sbody(c):
            i, tot, cutbin, cntab, _ = c
            h = hist[pl.ds(i * 16, 16)]
            cs = plsc.cumsum(h)
            ctot = _scalar(lax.reduce_max(cs, (0,)))
            a = (tot + ctot - cs) + h
            ge = a >= jnp.int32(_K)
            npos = _scalar(plsc.all_reduce_population_count(ge))
            crossed = npos > 0
            lstar = npos - 1
            al = _scalar(lax.reduce_max(
                jnp.where(lanes == lstar, a, 0), (0,)))
            hl = _scalar(lax.reduce_max(
                jnp.where(lanes == lstar, h, 0), (0,)))
            return (i - 1, tot + ctot,
                    jnp.where(crossed, i * 16 + lstar, cutbin),
                    jnp.where(crossed, al - hl, cntab),
                    crossed)

        _, _, cutbin, cntab, _ = lax.while_loop(
            _scond, _sbody,
            (jnp.int32(_NBINS // 16 - 1), jnp.int32(0), jnp.int32(0),
             jnp.int32(0), jnp.bool_(False)))

        u_lo = jnp.broadcast_to(lax.shift_left(cutbin, 21), (16,))
        u_hi = jnp.broadcast_to(lax.shift_left(cutbin + 1, 21), (16,))
        lo_f = plsc.bitcast(
            jnp.where(u_lo < 0, u_lo ^ jnp.int32(_IMIN), ~u_lo), jnp.float32)
        hi_f = plsc.bitcast(
            jnp.where(u_hi < 0, u_hi ^ jnp.int32(_IMIN), ~u_hi), jnp.float32)

        @plsc.parallel_loop(0, _CHUNKS, unroll=8,
                            carry=(jnp.zeros((16,), jnp.int32),
                                   jnp.zeros((16,), jnp.float32)))
        def _collect(i, c):
            offv, acc = c
            gv = g_v[r, pl.ds(i * 16, 16)]
            pv = p_v[r, pl.ds(i * 16, 16)]
            ge_lo = gv >= lo_f
            ge_hi = gv >= hi_f
            eq = jnp.logical_and(ge_lo, jnp.logical_not(ge_hi))
            d = pv - gv
            d2 = d * d
            acc = acc + jnp.where(ge_hi, d2, jnp.float32(0.0))
            off = _scalar(offv)
            plsc.store_compressed(collk.at[pl.ds(off, 16)], gv, mask=eq)
            plsc.store_compressed(collv.at[pl.ds(off, 16)], d2, mask=eq)
            return offv + plsc.all_reduce_population_count(eq), acc

        offv, acc = _collect
        off = _scalar(offv)
        ninf = jnp.full((16,), jnp.float32(-jnp.inf))
        collk[pl.ds(off, 16)] = ninf
        collv[pl.ds(off, 16)] = jnp.zeros((16,), jnp.float32)

        s0 = plsc.sort_key_val(collk[pl.ds(0, 16)], collv[pl.ds(0, 16)],
                               descending=True)
        ak, av = s0[0], s0[1]
        bk = ninf
        bv = jnp.zeros((16,), jnp.float32)

        def _mcond(c):
            return c[0] * 16 < off

        def _mbody(c):
            j, ak, av, bk, bv = c
            s = plsc.sort_key_val(collk[pl.ds(j * 16, 16)],
                                  collv[pl.ds(j * 16, 16)],
                                  descending=False)
            ck, cv = s[0], s[1]
            wa = ak >= ck
            hk = jnp.where(wa, ak, ck)
            hv = jnp.where(wa, av, cv)
            lk = jnp.where(wa, ck, ak)
            lv = jnp.where(wa, cv, av)
            s1 = plsc.sort_key_val(hk, hv, descending=True)
            s2 = plsc.sort_key_val(lk, lv, descending=False)
            wb = bk >= s2[0]
            h2k = jnp.where(wb, bk, s2[0])
            h2v = jnp.where(wb, bv, s2[1])
            s3 = plsc.sort_key_val(h2k, h2v, descending=True)
            return j + 1, s1[0], s1[1], s3[0], s3[1]

        _, ak, av, bk, bv = lax.while_loop(
            _mcond, _mbody, (jnp.int32(1), ak, av, bk, bv))

        r_need = jnp.int32(_K) - cntab
        msecut = (_scalar(lax.reduce_sum(
                      jnp.where(lanes < r_need, av, 0.0), (0,)))
                  + _scalar(lax.reduce_sum(
                      jnp.where(lanes + 16 < r_need, bv, 0.0), (0,))))
        msew = msew + _scalar(lax.reduce_sum(acc, (0,))) + msecut

    outv[...] = jnp.where(lanes == 0, msew, jnp.float32(0.0))
    pltpu.sync_copy(outv, out_hbm.at[wid])


def _sc_mse_partials(gamma, pred):
    mesh = plsc.VectorSubcoreMesh(core_axis_name="c", subcore_axis_name="s")
    run = pl.kernel(
        _sc_mse_body,
        out_type=jax.ShapeDtypeStruct((_NW, 16), jnp.float32),
        mesh=mesh,
        scratch_types=[
            pltpu.VMEM((_RPW, _N), jnp.float32),
            pltpu.VMEM((_RPW, _N), jnp.float32),
            pltpu.VMEM((_NBINS,), jnp.int32),
            pltpu.VMEM((_N + 32,), jnp.float32),
            pltpu.VMEM((_N + 32,), jnp.float32),
            pltpu.VMEM((16,), jnp.float32),
        ],
        compiler_params=pltpu.CompilerParams(needs_layout_passes=False),
    )
    return run(gamma, pred)


def _stats_body(p_ref, r_ref, g_ref, out_ref, acc_ref):
    i = pl.program_id(0)

    @pl.when(i == 0)
    def _init():
        acc_ref[0] = 0.0
        acc_ref[1] = 0.0

    g = g_ref[...]
    p = p_ref[...]
    inv_tau = jnp.float32(1.0 / _TAU)

    gmax = jnp.max(g, axis=1, keepdims=True)
    pmax = jnp.max(p, axis=1, keepdims=True)
    eg = jnp.exp((g - gmax) * inv_tau)
    ep = jnp.exp((p - pmax) * inv_tau)
    zg = jnp.sum(eg, axis=1, keepdims=True)
    zp = jnp.sum(ep, axis=1, keepdims=True)
    s_raw = jnp.sum(eg * (g - p), axis=1, keepdims=True)

    link = jnp.sum((r_ref[...] - gmax) ** 2)
    kl = jnp.sum(s_raw / (zg * _TAU) + (pmax - gmax) * inv_tau
                 + jnp.log(zp / zg))

    acc_ref[0] += link
    acc_ref[1] += kl

    @pl.when(i == _GRID - 1)
    def _fin():
        lane4 = lax.broadcasted_iota(jnp.int32, (1, 4), 1)
        out_ref[...] = jnp.where(
            lane4 == 0, acc_ref[0],
            jnp.where(lane4 == 1, acc_ref[1], 0.0))


def _combine_body(sc_ref, st_ref, out_ref):
    mse = jnp.sum(sc_ref[...][:, 0:1])
    st = st_ref[...]
    link = st[0, 0]
    kl = st[0, 1]
    total = (mse / jnp.float32(_B * _K)
             + _LAMBDA * link / _B
             + (_TAU * _TAU / _B) * kl)
    out_ref[...] = total.reshape((1, 1))


@jax.jit
def kernel(pred_logits, rsrp_pred, gamma_true):
    sc_out = _sc_mse_partials(gamma_true, pred_logits)
    stats = pl.pallas_call(
        _stats_body,
        grid=(_GRID,),
        in_specs=[
            pl.BlockSpec((_BLK, _N), lambda i: (i, 0)),
            pl.BlockSpec((_BLK, 1), lambda i: (i, 0)),
            pl.BlockSpec((_BLK, _N), lambda i: (i, 0)),
        ],
        out_specs=pl.BlockSpec((1, 4), lambda i: (0, 0)),
        out_shape=jax.ShapeDtypeStruct((1, 4), jnp.float32),
        scratch_shapes=[pltpu.SMEM((2,), jnp.float32)],
    )(pred_logits, rsrp_pred, gamma_true)
    out = pl.pallas_call(
        _combine_body,
        out_shape=jax.ShapeDtypeStruct((1, 1), jnp.float32),
    )(sc_out, stats)
    return out[0, 0]
